# baseline (speedup 1.0000x reference)
import functools

import jax
import jax.numpy as jnp
from jax import lax
from jax.experimental import pallas as pl
from jax.experimental.pallas import tpu as pltpu
from jax.experimental.pallas import tpu_sc as plsc

VOCAB = 10
HIDDEN = 4096
PRE = 16
BATCH = 4
SEQ = 8192

NC = 2
NS = 16
NW = NC * NS
ROWS_PER_W = BATCH * SEQ // NW
L = 16
GR = 8
NG = ROWS_PER_W // GR
NH = 4
CW = HIDDEN // NH
CG = CW // L
WPB = NW // BATCH


def _sc_embed(ids3, wef, prompt_embeddings):
    mesh = plsc.VectorSubcoreMesh(core_axis_name="c", subcore_axis_name="s")

    @functools.partial(
        pl.kernel,
        mesh=mesh,
        compiler_params=pltpu.CompilerParams(needs_layout_passes=False),
        out_type=jax.ShapeDtypeStruct((BATCH, PRE + SEQ, HIDDEN), jnp.float32),
        scratch_types=[
            pltpu.VMEM((ROWS_PER_W // L, L), jnp.int32),
            pltpu.VMEM((VOCAB * HIDDEN,), jnp.float32),
            pltpu.VMEM((GR, HIDDEN), jnp.float32),
            pltpu.VMEM((GR, HIDDEN), jnp.float32),
            pltpu.SemaphoreType.DMA,
            pltpu.SemaphoreType.DMA,
        ],
    )
    def k(ids_hbm, we_hbm, pe_hbm, out_hbm, idx_v, tab_v, st0, st1, sw0, sw1):
        wid = lax.axis_index("s") * NC + lax.axis_index("c")
        b = wid // WPB
        s0 = (wid % WPB) * ROWS_PER_W
        pltpu.sync_copy(ids_hbm.at[wid], idx_v)
        pltpu.sync_copy(we_hbm, tab_v)
        stages = (st0, st1)
        sws = (sw0, sw1)
        iota = lax.iota(jnp.int32, L)

        def write(r, pg):
            return pltpu.make_async_copy(
                stages[pg],
                out_hbm.at[b, pl.ds(PRE + s0 + r * GR, GR)],
                sws[pg],
            )

        def expand(r, pg):
            vec = idx_v[r // 2]
            lane0 = (r % 2) * GR

            def quarter(h, carry):
                coff = h * CW
                for j in range(GR):
                    sp = jnp.take(vec, jnp.full((L,), lane0 + j, jnp.int32))
                    fidx = sp * HIDDEN + coff + iota
                    for cg in range(CG):
                        val = plsc.load_gather(tab_v, [fidx])
                        stages[pg][j, pl.ds(coff + cg * L, L)] = val
                        fidx = fidx + L
                return carry

            lax.fori_loop(0, NH, quarter, 0)

        def body(r2, carry):
            for pg in range(2):
                r = 2 * r2 + pg

                @pl.when(r >= 2)
                def _():
                    write(r - 2, pg).wait()

                expand(r, pg)
                write(r, pg).start()
            return carry

        lax.fori_loop(0, NG // 2, body, 0)
        write(NG - 2, 0).wait()
        write(NG - 1, 1).wait()

        @pl.when(wid % WPB == 0)
        def _():
            for q in range(PRE // GR):
                pltpu.sync_copy(pe_hbm.at[pl.ds(q * GR, GR)], st0)
                pltpu.sync_copy(st0, out_hbm.at[b, pl.ds(q * GR, GR)])

    return k(ids3, wef, prompt_embeddings)


@jax.jit
def kernel(input_ids, word_embeddings, prompt_embeddings):
    ids3 = input_ids.astype(jnp.int32).reshape(NW, ROWS_PER_W // L, L)
    wef = word_embeddings.reshape(VOCAB * HIDDEN)
    return _sc_embed(ids3, wef, prompt_embeddings)


# speedup vs baseline: 9.4698x; 9.4698x over previous
import functools

import jax
import jax.numpy as jnp
from jax import lax
from jax.experimental import pallas as pl
from jax.experimental.pallas import tpu as pltpu
from jax.experimental.pallas import tpu_sc as plsc

VOCAB = 10
HIDDEN = 4096
PRE = 16
BATCH = 4
SEQ = 8192

NC = 2
NS = 16
NW = NC * NS
ROWS_PER_W = BATCH * SEQ // NW
L = 16
NCH = ROWS_PER_W // L
WPB = NW // BATCH


def _sc_embed(ids3, word_embeddings, prompt_embeddings):
    mesh = plsc.VectorSubcoreMesh(core_axis_name="c", subcore_axis_name="s")

    @functools.partial(
        pl.kernel,
        mesh=mesh,
        compiler_params=pltpu.CompilerParams(needs_layout_passes=False),
        out_type=jax.ShapeDtypeStruct((BATCH, PRE + SEQ, HIDDEN), jnp.float32),
        scratch_types=[
            pltpu.VMEM((NCH, L), jnp.int32),
            pltpu.VMEM((VOCAB, HIDDEN), jnp.float32),
            pltpu.VMEM((PRE // 2, HIDDEN), jnp.float32),
            pltpu.SemaphoreType.DMA,
        ],
    )
    def k(ids_hbm, we_hbm, pe_hbm, out_hbm, idx_v, tab_v, pe_v, sem):
        wid = lax.axis_index("s") * NC + lax.axis_index("c")
        b = wid // WPB
        s0 = (wid % WPB) * ROWS_PER_W
        pltpu.sync_copy(ids_hbm.at[wid], idx_v)
        pltpu.sync_copy(we_hbm, tab_v)
        lanes = lax.iota(jnp.int32, L)

        def body(c, carry):
            vec = idx_v[c]
            row0 = PRE + s0 + c * L
            for j in range(L):
                sid = jnp.sum(jnp.where(lanes == j, vec, 0))
                pltpu.make_async_copy(
                    tab_v.at[pl.ds(sid, 1)],
                    out_hbm.at[b, pl.ds(row0 + j, 1)],
                    sem,
                ).start()
            return carry

        lax.fori_loop(0, NCH, body, 0)
        pltpu.make_async_copy(
            out_hbm.at[b, pl.ds(PRE + s0, ROWS_PER_W)],
            out_hbm.at[b, pl.ds(PRE + s0, ROWS_PER_W)],
            sem,
        ).wait()

        @pl.when(wid % WPB == 0)
        def _():
            for q in range(2):
                pltpu.sync_copy(pe_hbm.at[pl.ds(q * (PRE // 2), PRE // 2)], pe_v)
                pltpu.sync_copy(pe_v, out_hbm.at[b, pl.ds(q * (PRE // 2), PRE // 2)])

    return k(ids3, word_embeddings, prompt_embeddings)


@jax.jit
def kernel(input_ids, word_embeddings, prompt_embeddings):
    ids3 = input_ids.astype(jnp.int32).reshape(NW, NCH, L)
    return _sc_embed(ids3, word_embeddings, prompt_embeddings)


# speedup vs baseline: 9.7081x; 1.0252x over previous
import functools

import jax
import jax.numpy as jnp
from jax import lax
from jax.experimental import pallas as pl
from jax.experimental.pallas import tpu as pltpu
from jax.experimental.pallas import tpu_sc as plsc

VOCAB = 10
HIDDEN = 4096
PRE = 16
BATCH = 4
SEQ = 8192

NC = 2
NS = 16
NW = NC * NS
ROWS_PER_W = BATCH * SEQ // NW
L = 16
NCH = ROWS_PER_W // L
WPB = NW // BATCH


def _sc_embed(ids3, word_embeddings, prompt_embeddings):
    mesh = plsc.VectorSubcoreMesh(core_axis_name="c", subcore_axis_name="s")

    @functools.partial(
        pl.kernel,
        mesh=mesh,
        compiler_params=pltpu.CompilerParams(needs_layout_passes=False),
        out_type=jax.ShapeDtypeStruct((BATCH, PRE + SEQ, HIDDEN), jnp.float32),
        scratch_types=[
            pltpu.VMEM((NCH, L), jnp.int32),
            pltpu.VMEM((VOCAB, HIDDEN), jnp.float32),
            pltpu.VMEM((PRE // WPB, HIDDEN), jnp.float32),
            pltpu.SemaphoreType.DMA,
            pltpu.SemaphoreType.DMA,
        ],
    )
    def k(ids_hbm, we_hbm, pe_hbm, out_hbm, idx_v, tab_v, pe_v, sem, sp):
        wid = lax.axis_index("s") * NC + lax.axis_index("c")
        b = wid // WPB
        q = wid % WPB
        s0 = q * ROWS_PER_W
        pltpu.sync_copy(ids_hbm.at[wid], idx_v)
        pltpu.sync_copy(we_hbm, tab_v)
        PQ = PRE // WPB
        pltpu.sync_copy(pe_hbm.at[pl.ds(q * PQ, PQ)], pe_v)
        pltpu.make_async_copy(
            pe_v, out_hbm.at[b, pl.ds(q * PQ, PQ)], sp
        ).start()
        lanes = lax.iota(jnp.int32, L)

        def body(c, carry):
            vec = idx_v[c]
            row0 = PRE + s0 + c * L
            for j in range(L):
                sid = jnp.sum(jnp.where(lanes == j, vec, 0))
                pltpu.make_async_copy(
                    tab_v.at[pl.ds(sid, 1)],
                    out_hbm.at[b, pl.ds(row0 + j, 1)],
                    sem,
                ).start()
            return carry

        lax.fori_loop(0, NCH, body, 0)
        pltpu.make_async_copy(
            out_hbm.at[b, pl.ds(PRE + s0, ROWS_PER_W)],
            out_hbm.at[b, pl.ds(PRE + s0, ROWS_PER_W)],
            sem,
        ).wait()
        pltpu.make_async_copy(
            pe_v, out_hbm.at[b, pl.ds(q * (PRE // WPB), PRE // WPB)], sp
        ).wait()

    return k(ids3, word_embeddings, prompt_embeddings)


@jax.jit
def kernel(input_ids, word_embeddings, prompt_embeddings):
    ids3 = input_ids.astype(jnp.int32).reshape(NW, NCH, L)
    return _sc_embed(ids3, word_embeddings, prompt_embeddings)


# speedup vs baseline: 9.9301x; 1.0229x over previous
import functools

import jax
import jax.numpy as jnp
from jax import lax
from jax.experimental import pallas as pl
from jax.experimental.pallas import tpu as pltpu
from jax.experimental.pallas import tpu_sc as plsc

VOCAB = 10
HIDDEN = 4096
PRE = 16
BATCH = 4
SEQ = 8192

NC = 2
NS = 16
NW = NC * NS
ROWS_PER_W = BATCH * SEQ // NW
L = 16
NCH = ROWS_PER_W // L
WPB = NW // BATCH


def _sc_embed(ids3, word_embeddings, prompt_embeddings):
    mesh = plsc.VectorSubcoreMesh(core_axis_name="c", subcore_axis_name="s")

    @functools.partial(
        pl.kernel,
        mesh=mesh,
        compiler_params=pltpu.CompilerParams(needs_layout_passes=False),
        out_type=jax.ShapeDtypeStruct((BATCH, PRE + SEQ, HIDDEN), jnp.float32),
        scratch_types=[
            pltpu.VMEM((NCH, L), jnp.int32),
            pltpu.VMEM((VOCAB, HIDDEN), jnp.float32),
            pltpu.VMEM((PRE // WPB, HIDDEN), jnp.float32),
            pltpu.SemaphoreType.DMA,
            pltpu.SemaphoreType.DMA,
        ],
    )
    def k(ids_hbm, we_hbm, pe_hbm, out_hbm, idx_v, tab_v, pe_v, sem, sp):
        wid = lax.axis_index("s") * NC + lax.axis_index("c")
        b = wid // WPB
        q = wid % WPB
        s0 = q * ROWS_PER_W
        pltpu.sync_copy(ids_hbm.at[wid], idx_v)
        pltpu.sync_copy(we_hbm, tab_v)
        PQ = PRE // WPB
        pltpu.sync_copy(pe_hbm.at[pl.ds(q * PQ, PQ)], pe_v)
        pltpu.make_async_copy(
            pe_v, out_hbm.at[b, pl.ds(q * PQ, PQ)], sp
        ).start()
        lanes = lax.iota(jnp.int32, L)

        def body(c, carry):
            vec = idx_v[c]
            row0 = PRE + s0 + c * L
            for j in range(L):
                sid = vec[j]
                pltpu.make_async_copy(
                    tab_v.at[pl.ds(sid, 1)],
                    out_hbm.at[b, pl.ds(row0 + j, 1)],
                    sem,
                ).start()
            return carry

        lax.fori_loop(0, NCH, body, 0)
        pltpu.make_async_copy(
            out_hbm.at[b, pl.ds(PRE + s0, ROWS_PER_W)],
            out_hbm.at[b, pl.ds(PRE + s0, ROWS_PER_W)],
            sem,
        ).wait()
        pltpu.make_async_copy(
            pe_v, out_hbm.at[b, pl.ds(q * (PRE // WPB), PRE // WPB)], sp
        ).wait()

    return k(ids3, word_embeddings, prompt_embeddings)


@jax.jit
def kernel(input_ids, word_embeddings, prompt_embeddings):
    ids3 = input_ids.astype(jnp.int32).reshape(NW, NCH, L)
    return _sc_embed(ids3, word_embeddings, prompt_embeddings)
